# baseline (speedup 1.0000x reference)
import functools

import jax
import jax.numpy as jnp
from jax import lax
from jax.experimental import pallas as pl
from jax.experimental.pallas import tpu as pltpu
from jax.experimental.pallas import tpu_sc as plsc

_NC = 2
_NS = 16
_NW = _NC * _NS
_G = 128
_SROWS = 16384


@functools.cache
def _build(N, D, C):
    K = C // _G
    per_w = N // _NW
    n_chunks = per_w // C
    mesh = plsc.VectorSubcoreMesh(core_axis_name="c", subcore_axis_name="s")

    @functools.partial(
        pl.kernel,
        out_type=jax.ShapeDtypeStruct((N, D), jnp.float32),
        mesh=mesh,
        scratch_types=[
            pltpu.VMEM((per_w // _G, _G), jnp.int32),
            pltpu.VMEM((per_w,), jnp.float32),
            pltpu.VMEM((2, C, D), jnp.float32),
            pltpu.VMEM_SHARED((_SROWS, D), jnp.float32),
            pltpu.SemaphoreType.DMA,
            pltpu.SemaphoreType.DMA,
            pltpu.SemaphoreType.DMA,
            pltpu.SemaphoreType.DMA,
        ],
        compiler_params=pltpu.CompilerParams(use_tc_tiling_on_sc=False),
    )
    def sc_kernel(x_hbm, val_hbm, table_hbm, out_hbm, idx_v, val_v, rows_v,
                  spmem, sem_g0, sem_g1, sem_o0, sem_o1):
        sems_g = (sem_g0, sem_g1)
        sems_o = (sem_o0, sem_o1)
        sid = lax.axis_index("s")
        wid = sid * _NC + lax.axis_index("c")
        base = wid * per_w
        pltpu.sync_copy(x_hbm.at[wid], idx_v)
        pltpu.sync_copy(val_hbm.at[pl.ds(base, per_w)], val_v)

        @pl.when(sid == 0)
        def _stage():
            pltpu.sync_copy(table_hbm.at[pl.ds(0, _SROWS)], spmem)

        plsc.subcore_barrier()

        def fire_gathers(c):
            b = c % 2
            descs = []
            for j in range(K):
                descs.append(pltpu.async_copy(
                    spmem.at[idx_v.at[c * K + j]],
                    rows_v.at[b, pl.ds(j * _G, _G)],
                    sems_g[b],
                ))
            return descs

        def scale_chunk(c):
            b = c % 2

            def grp_body(r, c2):
                val16 = val_v[pl.ds(c * C + r * 16, 16)]
                for j in range(16):
                    v = val16[j]
                    i = r * 16 + j
                    for h in range(D // 16):
                        sl = pl.ds(h * 16, 16)
                        rows_v[b, i, sl] = rows_v[b, i, sl] * v
                return c2

            lax.fori_loop(0, C // 16, grp_body, 0)

        g_descs = fire_gathers(0)
        out_descs = [None] * n_chunks
        for c in range(n_chunks):
            b = c % 2
            if c + 1 < n_chunks:
                if c >= 1:
                    out_descs[c - 1].wait()
                next_descs = fire_gathers(c + 1)
            for d in g_descs:
                d.wait()
            if c + 1 < n_chunks:
                g_descs = next_descs
            scale_chunk(c)
            out_descs[c] = pltpu.async_copy(
                rows_v.at[b], out_hbm.at[pl.ds(base + c * C, C)], sems_o[b])
        out_descs[n_chunks - 2].wait()
        out_descs[n_chunks - 1].wait()

    return sc_kernel


def kernel(x, x_val, table):
    B, NNZ = x.shape
    V, D = table.shape
    N = B * NNZ
    xf = (x % _SROWS).reshape(_NW, N // (_NW * _G), _G).astype(jnp.int32)
    vf = x_val.reshape(N)
    out = _build(N, D, 256)(xf, vf, table)
    return out.reshape(B, NNZ, D)


# speedup vs baseline: 1.1232x; 1.1232x over previous
import functools

import jax
import jax.numpy as jnp
from jax import lax
from jax.experimental import pallas as pl
from jax.experimental.pallas import tpu as pltpu
from jax.experimental.pallas import tpu_sc as plsc

_NC = 2
_NS = 16
_NW = _NC * _NS
_G = 128


@functools.cache
def _build(N, D, C):
    K = C // _G
    per_w = N // _NW
    n_chunks = per_w // C
    orows = C * D // 128
    mesh = plsc.VectorSubcoreMesh(core_axis_name="c", subcore_axis_name="s")

    @functools.partial(
        pl.kernel,
        out_type=jax.ShapeDtypeStruct((N * D // 128, 128), jnp.float32),
        mesh=mesh,
        scratch_types=[
            pltpu.VMEM((per_w,), jnp.int32),
            pltpu.VMEM((per_w,), jnp.int32),
            pltpu.VMEM((per_w,), jnp.float32),
            pltpu.VMEM((2, C, 128), jnp.float32),
            pltpu.VMEM((2, orows, 128), jnp.float32),
            pltpu.SemaphoreType.DMA,
            pltpu.SemaphoreType.DMA,
            pltpu.SemaphoreType.DMA,
            pltpu.SemaphoreType.DMA,
        ],
    )
    def sc_kernel(q_hbm, s_hbm, val_hbm, table_hbm, out_hbm,
                  q_v, s_v, val_v, quad_v, out_v,
                  sem_g0, sem_g1, sem_o0, sem_o1):
        wid = lax.axis_index("s") * _NC + lax.axis_index("c")
        base = pl.multiple_of(wid * per_w, 128)
        pltpu.sync_copy(q_hbm.at[pl.ds(base, per_w)], q_v)
        pltpu.sync_copy(s_hbm.at[pl.ds(base, per_w)], s_v)
        pltpu.sync_copy(val_hbm.at[pl.ds(base, per_w)], val_v)

        def gather_descs(c, b, sem):
            return [pltpu.make_async_copy(
                table_hbm.at[q_v.at[pl.ds((c * K + j) * _G, _G)]],
                quad_v.at[b, pl.ds(j * _G, _G)],
                sem,
            ) for j in range(K)]

        def per_parity(c, fn):
            @pl.when(c % 2 == 0)
            def _():
                fn(0)

            @pl.when(c % 2 == 1)
            def _():
                fn(1)

        def fire_gathers(c):
            sems = (sem_g0, sem_g1)
            per_parity(c, lambda b: [d.start() for d in
                                     gather_descs(c, b, sems[b])])

        def wait_gathers(c):
            sems = (sem_g0, sem_g1)
            per_parity(c, lambda b: [d.wait() for d in
                                     gather_descs(c, b, sems[b])])

        def out_desc(c, b, sem):
            dst = out_hbm.at[
                pl.ds(pl.multiple_of((base + c * C) * D // 128, 8), orows)]
            return pltpu.make_async_copy(out_v.at[b], dst, sem)

        def fire_out(c):
            sems = (sem_o0, sem_o1)
            per_parity(c, lambda b: out_desc(c, b, sems[b]).start())

        def wait_out(c):
            sems = (sem_o0, sem_o1)
            per_parity(c, lambda b: out_desc(c, b, sems[b]).wait())

        def scale_chunk(c):
            b = c % 2

            def grp_body(r, c2):
                s16 = s_v[pl.ds(c * C + r * 16, 16)]
                val16 = val_v[pl.ds(c * C + r * 16, 16)]
                off16 = s16 * D
                for j in range(16):
                    v = val16[j]
                    off = off16[j]
                    i = r * 16 + j
                    for h in range(D // 16):
                        seg = quad_v[b, i, pl.ds(off + h * 16, 16)]
                        out_v[b, 4 * r + (j * D + h * 16) // 128,
                              pl.ds((j * D + h * 16) % 128, 16)] = seg * v
                return c2

            lax.fori_loop(0, C // 16, grp_body, 0)

        fire_gathers(0)

        def chunk_body(c, carry):
            @pl.when(c < n_chunks - 1)
            def _():
                fire_gathers(c + 1)

            wait_gathers(c)

            @pl.when(c >= 2)
            def _():
                wait_out(c - 2)

            scale_chunk(c)
            fire_out(c)
            return carry

        lax.fori_loop(0, n_chunks, chunk_body, 0)
        wait_out(n_chunks - 2)
        wait_out(n_chunks - 1)

    return sc_kernel


def kernel(x, x_val, table):
    B, NNZ = x.shape
    V, D = table.shape
    N = B * NNZ
    rpq = 128 // D
    xi = x.astype(jnp.int32)
    q = (xi // rpq).reshape(N)
    s = (xi % rpq).reshape(N)
    vf = x_val.reshape(N)
    t4 = table.reshape(V // rpq, 128)
    out = _build(N, D, 256)(q, s, vf, t4)
    return out.reshape(B, NNZ, D)


# speedup vs baseline: 1.1240x; 1.0008x over previous
import functools

import jax
import jax.numpy as jnp
from jax import lax
from jax.experimental import pallas as pl
from jax.experimental.pallas import tpu as pltpu
from jax.experimental.pallas import tpu_sc as plsc

_NC = 2
_NS = 16
_NW = _NC * _NS


@functools.cache
def _build(B, NNZ, V, D):
    L = 16
    bw = B // _NW
    rpq = 128 // D
    shf = (rpq - 1).bit_length()
    dshf = (D - 1).bit_length()
    mesh = plsc.VectorSubcoreMesh(core_axis_name="c", subcore_axis_name="s")

    @functools.partial(
        pl.kernel,
        out_type=jax.ShapeDtypeStruct((NNZ, D, B), jnp.float32),
        mesh=mesh,
        scratch_types=[
            pltpu.VMEM((NNZ, bw), jnp.int32),
            pltpu.VMEM((NNZ, bw), jnp.float32),
            pltpu.VMEM((NNZ * bw,), jnp.int32),
            pltpu.VMEM((NNZ, bw), jnp.int32),
            pltpu.VMEM((2, bw, 128), jnp.float32),
            pltpu.VMEM((2, D, bw), jnp.float32),
            pltpu.SemaphoreType.DMA,
            pltpu.SemaphoreType.DMA,
            pltpu.SemaphoreType.DMA,
            pltpu.SemaphoreType.DMA,
        ],
        compiler_params=pltpu.CompilerParams(needs_layout_passes=False),
    )
    def sc_kernel(xt_hbm, vt_hbm, table_hbm, out_hbm,
                  x_v, val_v, q_v, off_v, quad_v, out_v,
                  sem_g0, sem_g1, sem_o0, sem_o1):
        wid = lax.axis_index("s") * _NC + lax.axis_index("c")
        b0 = pl.multiple_of(wid * bw, 128)
        pltpu.sync_copy(xt_hbm.at[:, pl.ds(b0, bw)], x_v)
        pltpu.sync_copy(vt_hbm.at[:, pl.ds(b0, bw)], val_v)

        def fmt_body(n, carry):
            for k in range(bw // L):
                x16 = x_v[n, pl.ds(k * L, L)]
                q_v[pl.ds(n * bw + k * L, L)] = lax.shift_right_logical(x16, shf)
                off_v[n, pl.ds(k * L, L)] = lax.shift_left(jnp.bitwise_and(x16, rpq - 1), dshf)
            return carry

        lax.fori_loop(0, NNZ, fmt_body, 0)

        def gather_desc(n, p, sem):
            return pltpu.make_async_copy(
                table_hbm.at[q_v.at[pl.ds(n * bw, bw)]],
                quad_v.at[p],
                sem,
            )

        def out_desc(n, p, sem):
            return pltpu.make_async_copy(
                out_v.at[p],
                out_hbm.at[n, :, pl.ds(b0, bw)],
                sem,
            )

        def per_parity(c, fn):
            @pl.when(c % 2 == 0)
            def _():
                fn(0)

            @pl.when(c % 2 == 1)
            def _():
                fn(1)

        sems_g = (sem_g0, sem_g1)
        sems_o = (sem_o0, sem_o1)

        def compute_p(n, p):
            row0 = lax.iota(jnp.int32, L)
            for k in range(bw // L):
                off16 = off_v[n, pl.ds(k * L, L)]
                val16 = val_v[n, pl.ds(k * L, L)]
                rows16 = row0 + k * L
                for c in range(D):
                    seg = plsc.load_gather(
                        quad_v.at[p], [rows16, off16 + c])
                    out_v[p, c, pl.ds(k * L, L)] = seg * val16

        def compute(n):
            per_parity(n, lambda p: compute_p(n, p))

        per_parity(0, lambda p: gather_desc(0, p, sems_g[p]).start())

        def n_body(n, carry):
            @pl.when(n < NNZ - 1)
            def _():
                per_parity(n + 1,
                           lambda p: gather_desc(n + 1, p, sems_g[p]).start())

            per_parity(n, lambda p: gather_desc(n, p, sems_g[p]).wait())

            @pl.when(n >= 2)
            def _():
                per_parity(n - 2,
                           lambda p: out_desc(n - 2, p, sems_o[p]).wait())

            compute(n)
            per_parity(n, lambda p: out_desc(n, p, sems_o[p]).start())
            return carry

        lax.fori_loop(0, NNZ, n_body, 0)
        per_parity(NNZ - 2, lambda p: out_desc(NNZ - 2, p, sems_o[p]).wait())
        per_parity(NNZ - 1, lambda p: out_desc(NNZ - 1, p, sems_o[p]).wait())

    return sc_kernel


def kernel(x, x_val, table):
    B, NNZ = x.shape
    V, D = table.shape
    rpq = 128 // D
    xt = jnp.transpose(x).astype(jnp.int32)
    vt = jnp.transpose(x_val)
    t4 = table.reshape(V // rpq, 128)
    out_t = _build(B, NNZ, V, D)(xt, vt, t4)
    return jnp.transpose(out_t, (2, 0, 1))
